# baseline (speedup 1.0000x reference)
import dataclasses

import jax
import jax.numpy as jnp
from jax import lax
from jax.experimental import pallas as pl
from jax.experimental.pallas import tpu as pltpu
from jax.experimental.pallas import tpu_sc as plsc

DIM = 64
BATCH = 16384
NC = 2
NS = 16
L = 16
NW = NC * NS
B_PER_W = BATCH // NW
CHUNK = 128
NCHUNK = B_PER_W // CHUNK


def _mf_body(uidx_hbm, iidx_hbm, uemb_hbm, iemb_hbm, ubias_hbm, ibias_hbm,
             out_hbm, uidx_v, iidx_v, u_v, i_v, ub_v, ib_v, part_v, out_v, sem):
    wid = lax.axis_index("s") * NC + lax.axis_index("c")
    row0 = wid * NCHUNK

    pltpu.sync_copy(uidx_hbm.at[pl.ds(row0, NCHUNK)], uidx_v)
    pltpu.sync_copy(iidx_hbm.at[pl.ds(row0, NCHUNK)], iidx_v)

    copies = []
    for c in range(NCHUNK):
        sl = pl.ds(c * CHUNK, CHUNK)
        copies.append(pltpu.async_copy(uemb_hbm.at[uidx_v.at[c]], u_v.at[sl], sem))
        copies.append(pltpu.async_copy(iemb_hbm.at[iidx_v.at[c]], i_v.at[sl], sem))
        copies.append(pltpu.async_copy(ubias_hbm.at[uidx_v.at[c]], ub_v.at[sl], sem))
        copies.append(pltpu.async_copy(ibias_hbm.at[iidx_v.at[c]], ib_v.at[sl], sem))
    for cp in copies:
        cp.wait()

    @pl.loop(0, B_PER_W)
    def _(r):
        acc = u_v[r, pl.ds(0, L)] * i_v[r, pl.ds(0, L)]
        for k in range(1, DIM // L):
            acc = acc + u_v[r, pl.ds(k * L, L)] * i_v[r, pl.ds(k * L, L)]
        part_v[pl.ds(r * L, L)] = acc

    iota = lax.iota(jnp.int32, L)

    @pl.loop(0, B_PER_W // L)
    def _(g):
        idx0 = g * (L * L) + iota * L
        acc = plsc.load_gather(part_v, [idx0])
        for d in range(1, L):
            acc = acc + plsc.load_gather(part_v, [idx0 + d])
        sl = pl.ds(g * L, L)
        out_v[sl] = acc + ub_v[sl] + ib_v[sl]

    pltpu.sync_copy(out_v, out_hbm.at[pl.ds(wid * B_PER_W, B_PER_W)])


def kernel(user, item, user_emb, item_emb, user_bias, item_bias):
    uidx = user.reshape(BATCH // CHUNK, CHUNK).astype(jnp.int32)
    iidx = item.reshape(BATCH // CHUNK, CHUNK).astype(jnp.int32)
    ub = user_bias.reshape(-1)
    ib = item_bias.reshape(-1)
    mesh = plsc.VectorSubcoreMesh(core_axis_name="c", subcore_axis_name="s")
    cp = pltpu.CompilerParams()
    if "needs_layout_passes" in pltpu.CompilerParams.__dataclass_fields__:
        cp = dataclasses.replace(cp, needs_layout_passes=False)
    if "use_tc_tiling_on_sc" in pltpu.CompilerParams.__dataclass_fields__:
        cp = dataclasses.replace(cp, use_tc_tiling_on_sc=False)
    mf = pl.kernel(
        _mf_body,
        out_type=jax.ShapeDtypeStruct((BATCH,), jnp.float32),
        mesh=mesh,
        compiler_params=cp,
        scratch_types=[
            pltpu.VMEM((NCHUNK, CHUNK), jnp.int32),
            pltpu.VMEM((NCHUNK, CHUNK), jnp.int32),
            pltpu.VMEM((B_PER_W, DIM), jnp.float32),
            pltpu.VMEM((B_PER_W, DIM), jnp.float32),
            pltpu.VMEM((B_PER_W,), jnp.float32),
            pltpu.VMEM((B_PER_W,), jnp.float32),
            pltpu.VMEM((B_PER_W * L,), jnp.float32),
            pltpu.VMEM((B_PER_W,), jnp.float32),
            pltpu.SemaphoreType.DMA,
        ],
    )
    return mf(uidx, iidx, user_emb, item_emb, ub, ib)
